# baseline (speedup 1.0000x reference)
import functools

import jax
import jax.numpy as jnp
from jax import lax
from jax.experimental import pallas as pl
from jax.experimental.pallas import tpu as pltpu
from jax.experimental.pallas import tpu_sc as plsc

ROWS = 32768
COLS = 64
NC = 2
NS = 16
L = 16

R_CHUNK = ROWS // NS
C_CHUNK = COLS // NC
UNROLL = 8

_mesh = plsc.VectorSubcoreMesh(core_axis_name="c", subcore_axis_name="s")


@functools.partial(
    pl.kernel,
    out_type=jax.ShapeDtypeStruct((ROWS, COLS), jnp.float32),
    mesh=_mesh,
    scratch_types=[
        pltpu.VMEM((R_CHUNK, C_CHUNK), jnp.float32),
        pltpu.VMEM((C_CHUNK,), jnp.float32),
        pltpu.VMEM((NS, C_CHUNK), jnp.float32),
        pltpu.MemorySpace.VMEM_SHARED((NS, C_CHUNK), jnp.float32),
    ],
    compiler_params=pltpu.CompilerParams(use_tc_tiling_on_sc=False),
)
def _sc_cumsum(x_hbm, out_hbm, buf, tv, at_buf, shared):
    c = lax.axis_index("c")
    s = lax.axis_index("s")
    r0 = s * R_CHUNK
    c0 = c * C_CHUNK

    pltpu.sync_copy(x_hbm.at[pl.ds(r0, R_CHUNK), pl.ds(c0, C_CHUNK)], buf)

    zero = jnp.zeros((L,), jnp.float32)

    def body_a(i, accs):
        a0, a1 = accs
        for u in range(UNROLL):
            r = i * UNROLL + u
            a0 = a0 + buf[r, pl.ds(0, L)]
            a1 = a1 + buf[r, pl.ds(L, L)]
        return (a0, a1)

    t0, t1 = lax.fori_loop(0, R_CHUNK // UNROLL, body_a, (zero, zero))

    tv[pl.ds(0, L)] = t0
    tv[pl.ds(L, L)] = t1
    pltpu.sync_copy(tv, shared.at[s])
    plsc.subcore_barrier()
    pltpu.sync_copy(shared, at_buf)

    def body_p(t, offs):
        o0, o1 = offs
        return (o0 + at_buf[t, pl.ds(0, L)], o1 + at_buf[t, pl.ds(L, L)])

    minus1 = jnp.full((L,), -1.0, jnp.float32)
    o0, o1 = lax.fori_loop(0, s, body_p, (minus1, minus1))

    def body_b(i, accs):
        a0, a1 = accs
        for u in range(UNROLL):
            r = i * UNROLL + u
            a0 = a0 + buf[r, pl.ds(0, L)]
            a1 = a1 + buf[r, pl.ds(L, L)]
            buf[r, pl.ds(0, L)] = a0
            buf[r, pl.ds(L, L)] = a1
        return (a0, a1)

    lax.fori_loop(0, R_CHUNK // UNROLL, body_b, (o0, o1))

    pltpu.sync_copy(buf, out_hbm.at[pl.ds(r0, R_CHUNK), pl.ds(c0, C_CHUNK)])


def kernel(mask_i):
    return _sc_cumsum(mask_i)


# speedup vs baseline: 1.0280x; 1.0280x over previous
import functools

import jax
import jax.numpy as jnp
from jax import lax
from jax.experimental import pallas as pl
from jax.experimental.pallas import tpu as pltpu
from jax.experimental.pallas import tpu_sc as plsc

ROWS = 32768
COLS = 64
NC = 2
NS = 16
NT = NC * NS
L = 16
NG = COLS // L

R_CHUNK = ROWS // NT
SUB = 2
R_SUB = R_CHUNK // SUB
PAD = 8
UNROLL = 8

_mesh = plsc.VectorSubcoreMesh(core_axis_name="c", subcore_axis_name="s")
_params = pltpu.CompilerParams(use_tc_tiling_on_sc=True)


def _wid():
    return lax.axis_index("c") * NS + lax.axis_index("s")


@functools.partial(
    pl.kernel,
    out_type=jax.ShapeDtypeStruct((NT * PAD, COLS), jnp.float32),
    mesh=_mesh,
    scratch_types=[
        pltpu.VMEM((R_SUB, COLS), jnp.float32),
        pltpu.VMEM((PAD, COLS), jnp.float32),
    ],
    compiler_params=_params,
)
def _sc_totals(x_hbm, tot_hbm, buf, tv):
    w = _wid()
    r0 = w * R_CHUNK

    def body_a(i, accs):
        accs = list(accs)
        for u in range(UNROLL):
            r = i * UNROLL + u
            for g in range(NG):
                accs[g] = accs[g] + buf[r, pl.ds(g * L, L)]
        return tuple(accs)

    zero = jnp.zeros((L,), jnp.float32)
    tots = (zero,) * NG
    for k in range(SUB):
        pltpu.sync_copy(x_hbm.at[pl.ds(r0 + k * R_SUB, R_SUB), :], buf)
        tots = lax.fori_loop(0, R_SUB // UNROLL, body_a, tots)

    for g in range(NG):
        for p in range(PAD):
            tv[p, pl.ds(g * L, L)] = tots[g]
    pltpu.sync_copy(tv, tot_hbm.at[pl.ds(w * PAD, PAD), :])


@functools.partial(
    pl.kernel,
    out_type=jax.ShapeDtypeStruct((ROWS, COLS), jnp.float32),
    mesh=_mesh,
    scratch_types=[
        pltpu.VMEM((R_SUB, COLS), jnp.float32),
        pltpu.VMEM((NT * PAD, COLS), jnp.float32),
    ],
    compiler_params=_params,
)
def _sc_scan(x_hbm, tot_hbm, out_hbm, buf, tb):
    w = _wid()
    r0 = w * R_CHUNK
    pltpu.sync_copy(tot_hbm, tb)

    def body_p(t, offs):
        return tuple(offs[g] + tb[t * PAD, pl.ds(g * L, L)] for g in range(NG))

    minus1 = jnp.full((L,), -1.0, jnp.float32)
    offs = lax.fori_loop(0, w, body_p, (minus1,) * NG)

    def body_b(i, accs):
        accs = list(accs)
        for u in range(UNROLL):
            r = i * UNROLL + u
            for g in range(NG):
                accs[g] = accs[g] + buf[r, pl.ds(g * L, L)]
                buf[r, pl.ds(g * L, L)] = accs[g]
        return tuple(accs)

    for k in range(SUB):
        pltpu.sync_copy(x_hbm.at[pl.ds(r0 + k * R_SUB, R_SUB), :], buf)
        offs = lax.fori_loop(0, R_SUB // UNROLL, body_b, offs)
        pltpu.sync_copy(buf, out_hbm.at[pl.ds(r0 + k * R_SUB, R_SUB), :])


def kernel(mask_i):
    tot = _sc_totals(mask_i)
    return _sc_scan(mask_i, tot)


# speedup vs baseline: 1.8264x; 1.7766x over previous
import functools

import jax
import jax.numpy as jnp
from jax import lax
from jax.experimental import pallas as pl
from jax.experimental.pallas import tpu as pltpu
from jax.experimental.pallas import tpu_sc as plsc

ROWS = 32768
COLS = 64
L = 16

RG = 8
NGRP = COLS // RG
KCH = 4
C_CHUNK = ROWS // KCH
NVREG = C_CHUNK // L

_mesh = plsc.VectorSubcoreMesh(core_axis_name="c", subcore_axis_name="s",
                               num_cores=2, num_subcores=16)
_params = pltpu.CompilerParams(use_tc_tiling_on_sc=True,
                               needs_layout_passes=False)

_GDN = lax.GatherDimensionNumbers(
    offset_dims=(), collapsed_slice_dims=(0,), start_index_map=(0,))


def _bcast_last(y):
    idx = jnp.full((L, 1), L - 1, jnp.int32)
    return lax.gather(y, idx, _GDN, (1,),
                      mode=lax.GatherScatterMode.PROMISE_IN_BOUNDS)


def _sc_body(x_hbm, out_hbm, buf, tv, at_buf, shared):
    c = lax.axis_index("c")
    s = lax.axis_index("s")
    g_local = s // KCH
    k = s % KCH
    r0 = (c * (NGRP // 2) + g_local) * RG
    c0 = k * C_CHUNK

    pltpu.sync_copy(x_hbm.at[pl.ds(r0, RG), pl.ds(c0, C_CHUNK)], buf)

    zero = jnp.zeros((L,), jnp.float32)

    def body_a(j, accs):
        return tuple(accs[r] + buf[r, pl.ds(j * L, L)] for r in range(RG))

    tot = lax.fori_loop(0, NVREG, body_a, (zero,) * RG)
    for r in range(RG):
        tv[r, pl.ds(0, L)] = jnp.full((L,), jnp.sum(tot[r]), jnp.float32)

    pltpu.sync_copy(tv, shared.at[s])
    plsc.subcore_barrier()
    pltpu.sync_copy(shared, at_buf)

    offs = [jnp.full((L,), -1.0, jnp.float32) for _ in range(RG)]
    for kp in range(KCH - 1):
        m = (kp < k).astype(jnp.float32)
        src = g_local * KCH + kp
        for r in range(RG):
            offs[r] = offs[r] + at_buf[src, r, pl.ds(0, L)] * m

    def body_b(j, carrys):
        new = []
        for r in range(RG):
            x = buf[r, pl.ds(j * L, L)]
            y = plsc.cumsum(x) + carrys[r]
            buf[r, pl.ds(j * L, L)] = y
            new.append(_bcast_last(y))
        return tuple(new)

    lax.fori_loop(0, NVREG, body_b, tuple(offs))

    pltpu.sync_copy(buf, out_hbm.at[pl.ds(r0, RG), pl.ds(c0, C_CHUNK)])


def _build(interpret=False):
    return pl.kernel(
        _sc_body,
        out_type=jax.ShapeDtypeStruct((COLS, ROWS), jnp.float32),
        mesh=_mesh,
        scratch_types=[
            pltpu.VMEM((RG, C_CHUNK), jnp.float32),
            pltpu.VMEM((RG, 128), jnp.float32),
            pltpu.VMEM((16, RG, 128), jnp.float32),
            pltpu.MemorySpace.VMEM_SHARED((16, RG, 128), jnp.float32),
        ],
        compiler_params=_params,
        interpret=interpret,
    )


_sc_cumsum_t = _build()


def kernel(mask_i):
    return _sc_cumsum_t(mask_i.T).T


# speedup vs baseline: 2.2012x; 1.2052x over previous
import functools

import jax
import jax.numpy as jnp
from jax import lax
from jax.experimental import pallas as pl
from jax.experimental.pallas import tpu as pltpu
from jax.experimental.pallas import tpu_sc as plsc

ROWS = 32768
COLS = 64
L = 16

RG = 8
NGRP = COLS // RG
KCH = 4
C_CHUNK = ROWS // KCH
NVREG = C_CHUNK // L

_mesh = plsc.VectorSubcoreMesh(core_axis_name="c", subcore_axis_name="s",
                               num_cores=2, num_subcores=16)
_params = pltpu.CompilerParams(use_tc_tiling_on_sc=True,
                               needs_layout_passes=False)

_GDN = lax.GatherDimensionNumbers(
    offset_dims=(), collapsed_slice_dims=(0,), start_index_map=(0,))


def _bcast_last(y):
    idx = jnp.full((L, 1), L - 1, jnp.int32)
    return lax.gather(y, idx, _GDN, (1,),
                      mode=lax.GatherScatterMode.PROMISE_IN_BOUNDS)


NPAN = 8
C_PAN = C_CHUNK // NPAN
NV_PAN = C_PAN // L


def _sc_body(x_hbm, out_hbm, buf, tv, at_buf, shared, in_sems, out_sems):
    c = lax.axis_index("c")
    s = lax.axis_index("s")
    g_local = s // KCH
    k = s % KCH
    r0 = (c * (NGRP // 2) + g_local) * RG
    c0 = k * C_CHUNK

    in_copies = []
    for p in range(NPAN):
        cp = pltpu.make_async_copy(
            x_hbm.at[pl.ds(r0, RG), pl.ds(c0 + p * C_PAN, C_PAN)],
            buf.at[:, pl.ds(p * C_PAN, C_PAN)],
            in_sems.at[p])
        cp.start()
        in_copies.append(cp)

    zero = jnp.zeros((L,), jnp.float32)
    tot = (zero,) * RG
    for p in range(NPAN):
        in_copies[p].wait()

        def body_a(j, accs):
            return tuple(accs[r] + buf[r, pl.ds(p * C_PAN + j * L, L)]
                         for r in range(RG))

        tot = lax.fori_loop(0, NV_PAN, body_a, tot)

    for r in range(RG):
        tv[r, pl.ds(0, L)] = jnp.full((L,), jnp.sum(tot[r]), jnp.float32)

    pltpu.sync_copy(tv, shared.at[s])
    plsc.subcore_barrier()
    pltpu.sync_copy(shared, at_buf)

    offs = [jnp.full((L,), -1.0, jnp.float32) for _ in range(RG)]
    for kp in range(KCH - 1):
        m = (kp < k).astype(jnp.float32)
        src = g_local * KCH + kp
        for r in range(RG):
            offs[r] = offs[r] + at_buf[src, r, pl.ds(0, L)] * m

    carrys = tuple(offs)
    out_copies = []
    for p in range(NPAN):

        def body_b(j, cs):
            new = []
            for r in range(RG):
                x = buf[r, pl.ds(p * C_PAN + j * L, L)]
                y = plsc.cumsum(x) + cs[r]
                buf[r, pl.ds(p * C_PAN + j * L, L)] = y
                new.append(_bcast_last(y))
            return tuple(new)

        carrys = lax.fori_loop(0, NV_PAN, body_b, carrys)
        cp = pltpu.make_async_copy(
            buf.at[:, pl.ds(p * C_PAN, C_PAN)],
            out_hbm.at[pl.ds(r0, RG), pl.ds(c0 + p * C_PAN, C_PAN)],
            out_sems.at[p])
        cp.start()
        out_copies.append(cp)

    for cp in out_copies:
        cp.wait()


def _build(interpret=False):
    return pl.kernel(
        _sc_body,
        out_type=jax.ShapeDtypeStruct((COLS, ROWS), jnp.float32),
        mesh=_mesh,
        scratch_types=[
            pltpu.VMEM((RG, C_CHUNK), jnp.float32),
            pltpu.VMEM((RG, 128), jnp.float32),
            pltpu.VMEM((16, RG, 128), jnp.float32),
            pltpu.MemorySpace.VMEM_SHARED((16, RG, 128), jnp.float32),
            pltpu.SemaphoreType.DMA((NPAN,)),
            pltpu.SemaphoreType.DMA((NPAN,)),
        ],
        compiler_params=_params,
        interpret=interpret,
    )


_sc_cumsum_t = _build()


def kernel(mask_i):
    return _sc_cumsum_t(mask_i.T).T
